# Initial kernel scaffold; baseline (speedup 1.0000x reference)
import functools

import jax
import jax.numpy as jnp
from jax import lax
from jax.experimental import pallas as pl
from jax.experimental.pallas import tpu as pltpu
from jax.experimental.pallas import tpu_sc as plsc

N = 10000
E = 320000
D = 128
B = 64

NC = 2
NS = 16
NW = NC * NS

NP = 10112
PAD_NODE = 10000
CH = 128
EPW = 10112
NCHUNK = EPW // CH
E_PAD = EPW * NW
RPT = NP // NS
DR = 80

_mesh = plsc.VectorSubcoreMesh(core_axis_name="c", subcore_axis_name="s")

_f32 = jnp.float32


def _deg_body(dst_hbm, zeros_hbm, deg0_hbm, deg1_hbm,
              dstv, deg_local, idv, sem, deg_sh):
    cid = lax.axis_index("c")
    sid = lax.axis_index("s")
    wid = cid * NS + sid

    pltpu.sync_copy(zeros_hbm.at[pl.ds(0, DR)], deg_local)
    pltpu.sync_copy(zeros_hbm.at[pl.ds(0, DR // NS)],
                    deg_sh.at[pl.ds(sid * (DR // NS), DR // NS)])
    for k in range(DR // 16):
        idv[pl.ds(k * 16, 16)] = lax.iota(jnp.int32, 16) + (k * 16)
    plsc.subcore_barrier()

    ones = jnp.full((16,), 1.0, dtype=_f32)
    base = wid * EPW

    @pl.loop(0, NCHUNK)
    def _chunks(i):
        pltpu.sync_copy(dst_hbm.at[pl.ds(base + i * CH, CH)], dstv)
        for k in range(CH // 16):
            idx = dstv[pl.ds(k * 16, 16)]
            row = lax.shift_right_logical(idx, 7)
            col = lax.bitwise_and(idx, 127)
            plsc.addupdate_scatter(deg_local, [row, col], ones)

    pltpu.sync_copy(deg_local, deg_sh.at[idv], add=True)
    plsc.subcore_barrier()

    rpt = DR // NS
    sl = pl.ds(sid * rpt, rpt)

    @pl.when(cid == 0)
    def _():
        pltpu.sync_copy(deg_sh.at[sl], deg0_hbm.at[sl])

    @pl.when(cid == 1)
    def _():
        pltpu.sync_copy(deg_sh.at[sl], deg1_hbm.at[sl])


_deg_kernel = functools.partial(
    pl.kernel, _deg_body, mesh=_mesh,
    out_type=[jax.ShapeDtypeStruct((DR, 128), _f32)] * 2,
    scratch_types=[
        pltpu.VMEM((CH,), jnp.int32),
        pltpu.VMEM((DR, 128), _f32),
        pltpu.VMEM((DR,), jnp.int32),
        pltpu.SemaphoreType.DMA,
        pltpu.VMEM_SHARED((DR, 128), _f32),
    ],
)()


def _agg_body(src_hbm, dst_hbm, y_hbm, zeros_hbm, t0_hbm, t1_hbm,
              srcv, dstv, rows, sem, t_sh):
    cid = lax.axis_index("c")
    sid = lax.axis_index("s")
    wid = cid * NS + sid

    row0 = sid * RPT
    init_src = (y_hbm, zeros_hbm)
    for c in range(NC):
        @pl.when(cid == c)
        def _():
            pltpu.sync_copy(init_src[c].at[pl.ds(row0, RPT)],
                            t_sh.at[pl.ds(row0, RPT)])
    plsc.subcore_barrier()

    base = wid * EPW

    @pl.loop(0, NCHUNK)
    def _chunks(i):
        off = base + i * CH
        pltpu.sync_copy(src_hbm.at[pl.ds(off, CH)], srcv)
        pltpu.sync_copy(dst_hbm.at[pl.ds(off, CH)], dstv)
        pltpu.async_copy(y_hbm.at[srcv], rows, sem).wait()
        pltpu.sync_copy(rows, t_sh.at[dstv], add=True)

    plsc.subcore_barrier()

    out = (t0_hbm, t1_hbm)
    for c in range(NC):
        @pl.when(cid == c)
        def _():
            pltpu.sync_copy(t_sh.at[pl.ds(row0, RPT)],
                            out[c].at[pl.ds(row0, RPT)])


_agg_kernel = functools.partial(
    pl.kernel, _agg_body, mesh=_mesh,
    out_type=[jax.ShapeDtypeStruct((NP, 128), _f32)] * 2,
    scratch_types=[
        pltpu.VMEM((CH,), jnp.int32),
        pltpu.VMEM((CH,), jnp.int32),
        pltpu.VMEM((CH, 128), _f32),
        pltpu.SemaphoreType.DMA,
        pltpu.VMEM_SHARED((NP, 128), _f32),
    ],
)()


def _prescale_body(x_ref, d0_ref, d1_ref, y_ref, isr_ref):
    deg = d0_ref[...] + d1_ref[...] + 1.0
    rows = lax.broadcasted_iota(jnp.int32, (NP, 1), 0)
    isr = jnp.where(rows < N, lax.rsqrt(deg), 0.0)
    isr_ref[...] = isr
    y_ref[...] = x_ref[...] * isr


_prescale = pl.pallas_call(
    _prescale_body,
    out_shape=[jax.ShapeDtypeStruct((NP, 128), _f32),
               jax.ShapeDtypeStruct((NP, 1), _f32)],
)


def _layer_body(t0_ref, t1_ref, isr_ref, w_ref, b_ref, y_ref):
    isr = isr_ref[...]
    h = (t0_ref[...] + t1_ref[...]) * isr
    x = jax.nn.relu(jnp.dot(h, w_ref[...], preferred_element_type=_f32)
                    + b_ref[...])
    y_ref[...] = x * isr


_layer = pl.pallas_call(
    _layer_body,
    out_shape=jax.ShapeDtypeStruct((NP, 128), _f32),
)


def _final_body(t0_ref, t1_ref, isr_ref, w_ref, b_ref, gid_ref, xa_ref,
                wd1a_ref, wd1b_ref, bd1_ref, wd2_ref, bd2_ref,
                wout_ref, bout_ref, out_ref):
    h = (t0_ref[...] + t1_ref[...]) * isr_ref[...]
    x3 = jax.nn.relu(jnp.dot(h, w_ref[...], preferred_element_type=_f32)
                     + b_ref[...])
    gsel = lax.broadcasted_iota(jnp.int32, (B, NP), 0)
    mask = jnp.where(gsel == gid_ref[...], 1.0, 0.0)
    r = jnp.dot(mask, x3, preferred_element_type=_f32)
    h1 = jax.nn.relu(
        jnp.dot(r, wd1a_ref[...], preferred_element_type=_f32)
        + jnp.dot(xa_ref[...], wd1b_ref[...], preferred_element_type=_f32)
        + bd1_ref[...])
    h2 = jax.nn.relu(jnp.dot(h1, wd2_ref[...], preferred_element_type=_f32)
                     + bd2_ref[...])
    out_ref[...] = (jnp.dot(h2, wout_ref[...], preferred_element_type=_f32)
                    + bout_ref[...])


_final = pl.pallas_call(
    _final_body,
    out_shape=jax.ShapeDtypeStruct((B, 1), _f32),
)


def kernel(x_mol, edge_index, graph_ids, x_adduct,
           Wg1, bg1, Wg2, bg2, Wg3, bg3,
           Wd1, bd1, Wd2, bd2, Wout, bout):
    x_pad = jnp.pad(x_mol, ((0, NP - N), (0, 0)))
    src = jnp.pad(edge_index[0], (0, E_PAD - E), constant_values=PAD_NODE)
    dst = jnp.pad(edge_index[1], (0, E_PAD - E), constant_values=PAD_NODE)
    gid = jnp.pad(graph_ids, (0, NP - N), constant_values=B)[None, :]
    zeros = jnp.zeros((NP, 128), dtype=_f32)

    deg0, deg1 = _deg_kernel(dst, zeros)
    d0 = deg0.reshape(-1)[:NP, None]
    d1 = deg1.reshape(-1)[:NP, None]
    y1, isr = _prescale(x_pad, d0, d1)

    t0, t1 = _agg_kernel(src, dst, y1, zeros)
    y2 = _layer(t0, t1, isr, Wg1, bg1[None, :])
    t0, t1 = _agg_kernel(src, dst, y2, zeros)
    y3 = _layer(t0, t1, isr, Wg2, bg2[None, :])
    t0, t1 = _agg_kernel(src, dst, y3, zeros)

    return _final(t0, t1, isr, Wg3, bg3[None, :], gid,
                  x_adduct.astype(_f32),
                  Wd1[:D], Wd1[D:], bd1[None, :],
                  Wd2, bd2[None, :], Wout, bout[None, :])


# speedup vs baseline: 7.4155x; 7.4155x over previous
import functools

import jax
import jax.numpy as jnp
from jax import lax
from jax.experimental import pallas as pl
from jax.experimental.pallas import tpu as pltpu
from jax.experimental.pallas import tpu_sc as plsc

N = 10000
E = 320000
D = 128
B = 64

NC = 2
NS = 16
NW = NC * NS

NP = 10112
PAD_NODE = 10000
CH = 128
EPW = 10112
NCHUNK = EPW // CH
E_PAD = EPW * NW
RPT = NP // NS
DR = 80

_mesh = plsc.VectorSubcoreMesh(core_axis_name="c", subcore_axis_name="s")

_f32 = jnp.float32


def _deg_body(dst_hbm, ones_hbm, zeros_hbm, deg0_hbm, deg1_hbm,
              dstv, ones_rows, sem, deg_sh):
    cid = lax.axis_index("c")
    sid = lax.axis_index("s")
    wid = cid * NS + sid
    row0 = sid * RPT

    pltpu.sync_copy(ones_hbm, ones_rows)
    pltpu.sync_copy(zeros_hbm.at[pl.ds(row0, RPT)],
                    deg_sh.at[pl.ds(row0, RPT)])
    plsc.subcore_barrier()

    base = wid * EPW

    @pl.loop(0, NCHUNK)
    def _chunks(i):
        pltpu.sync_copy(dst_hbm.at[pl.ds(base + i * CH, CH)], dstv)
        pltpu.sync_copy(ones_rows, deg_sh.at[dstv], add=True)

    plsc.subcore_barrier()

    out = (deg0_hbm, deg1_hbm)
    for c in range(NC):
        @pl.when(cid == c)
        def _():
            pltpu.sync_copy(deg_sh.at[pl.ds(row0, RPT)],
                            out[c].at[pl.ds(row0, RPT)])


_deg_kernel = functools.partial(
    pl.kernel, _deg_body, mesh=_mesh,
    out_type=[jax.ShapeDtypeStruct((NP, 128), _f32)] * 2,
    scratch_types=[
        pltpu.VMEM((CH,), jnp.int32),
        pltpu.VMEM((CH, 128), _f32),
        pltpu.SemaphoreType.DMA,
        pltpu.VMEM_SHARED((NP, 128), _f32),
    ],
)()


def _agg_body(src_hbm, dst_hbm, y_hbm, zeros_hbm, t0_hbm, t1_hbm,
              srcv, dstv, rows, sem, t_sh):
    cid = lax.axis_index("c")
    sid = lax.axis_index("s")
    wid = cid * NS + sid

    row0 = sid * RPT
    init_src = (y_hbm, zeros_hbm)
    for c in range(NC):
        @pl.when(cid == c)
        def _():
            pltpu.sync_copy(init_src[c].at[pl.ds(row0, RPT)],
                            t_sh.at[pl.ds(row0, RPT)])
    plsc.subcore_barrier()

    base = wid * EPW

    @pl.loop(0, NCHUNK)
    def _chunks(i):
        off = base + i * CH
        pltpu.sync_copy(src_hbm.at[pl.ds(off, CH)], srcv)
        pltpu.sync_copy(dst_hbm.at[pl.ds(off, CH)], dstv)
        pltpu.async_copy(y_hbm.at[srcv], rows, sem).wait()
        pltpu.sync_copy(rows, t_sh.at[dstv], add=True)

    plsc.subcore_barrier()

    out = (t0_hbm, t1_hbm)
    for c in range(NC):
        @pl.when(cid == c)
        def _():
            pltpu.sync_copy(t_sh.at[pl.ds(row0, RPT)],
                            out[c].at[pl.ds(row0, RPT)])


_agg_kernel = functools.partial(
    pl.kernel, _agg_body, mesh=_mesh,
    out_type=[jax.ShapeDtypeStruct((NP, 128), _f32)] * 2,
    scratch_types=[
        pltpu.VMEM((CH,), jnp.int32),
        pltpu.VMEM((CH,), jnp.int32),
        pltpu.VMEM((CH, 128), _f32),
        pltpu.SemaphoreType.DMA,
        pltpu.VMEM_SHARED((NP, 128), _f32),
    ],
)()


def _prescale_body(x_ref, d0_ref, d1_ref, y_ref, isr_ref):
    deg = d0_ref[...] + d1_ref[...] + 1.0
    rows = lax.broadcasted_iota(jnp.int32, (NP, 128), 0)
    isr = jnp.where(rows < N, lax.rsqrt(deg), 0.0)
    isr_ref[...] = isr
    y_ref[...] = x_ref[...] * isr


_prescale = pl.pallas_call(
    _prescale_body,
    out_shape=[jax.ShapeDtypeStruct((NP, 128), _f32),
               jax.ShapeDtypeStruct((NP, 128), _f32)],
)


def _layer_body(t0_ref, t1_ref, isr_ref, w_ref, b_ref, y_ref):
    isr = isr_ref[...]
    h = (t0_ref[...] + t1_ref[...]) * isr
    x = jax.nn.relu(jnp.dot(h, w_ref[...], preferred_element_type=_f32)
                    + b_ref[...])
    y_ref[...] = x * isr


_layer = pl.pallas_call(
    _layer_body,
    out_shape=jax.ShapeDtypeStruct((NP, 128), _f32),
)


def _final_body(t0_ref, t1_ref, isr_ref, w_ref, b_ref, gid_ref, xa_ref,
                wd1a_ref, wd1b_ref, bd1_ref, wd2_ref, bd2_ref,
                wout_ref, bout_ref, out_ref):
    h = (t0_ref[...] + t1_ref[...]) * isr_ref[...]
    x3 = jax.nn.relu(jnp.dot(h, w_ref[...], preferred_element_type=_f32)
                     + b_ref[...])
    gsel = lax.broadcasted_iota(jnp.int32, (B, NP), 0)
    mask = jnp.where(gsel == gid_ref[...], 1.0, 0.0)
    r = jnp.dot(mask, x3, preferred_element_type=_f32)
    h1 = jax.nn.relu(
        jnp.dot(r, wd1a_ref[...], preferred_element_type=_f32)
        + jnp.dot(xa_ref[...], wd1b_ref[...], preferred_element_type=_f32)
        + bd1_ref[...])
    h2 = jax.nn.relu(jnp.dot(h1, wd2_ref[...], preferred_element_type=_f32)
                     + bd2_ref[...])
    out_ref[...] = (jnp.dot(h2, wout_ref[...], preferred_element_type=_f32)
                    + bout_ref[...])


_final = pl.pallas_call(
    _final_body,
    out_shape=jax.ShapeDtypeStruct((B, 1), _f32),
)


def kernel(x_mol, edge_index, graph_ids, x_adduct,
           Wg1, bg1, Wg2, bg2, Wg3, bg3,
           Wd1, bd1, Wd2, bd2, Wout, bout):
    x_pad = jnp.pad(x_mol, ((0, NP - N), (0, 0)))
    src = jnp.pad(edge_index[0], (0, E_PAD - E), constant_values=PAD_NODE)
    dst = jnp.pad(edge_index[1], (0, E_PAD - E), constant_values=PAD_NODE)
    gid = jnp.pad(graph_ids, (0, NP - N), constant_values=B)[None, :]
    zeros = jnp.zeros((NP, 128), dtype=_f32)

    ones2d = jnp.ones((CH, 128), dtype=_f32)
    deg0, deg1 = _deg_kernel(dst, ones2d, zeros)
    y1, isr = _prescale(x_pad, deg0, deg1)

    t0, t1 = _agg_kernel(src, dst, y1, zeros)
    y2 = _layer(t0, t1, isr, Wg1, bg1[None, :])
    t0, t1 = _agg_kernel(src, dst, y2, zeros)
    y3 = _layer(t0, t1, isr, Wg2, bg2[None, :])
    t0, t1 = _agg_kernel(src, dst, y3, zeros)

    return _final(t0, t1, isr, Wg3, bg3[None, :], gid,
                  x_adduct.astype(_f32),
                  Wd1[:D], Wd1[D:], bd1[None, :],
                  Wd2, bd2[None, :], Wout, bout[None, :])
